# baseline (speedup 1.0000x reference)
import functools

import jax
import jax.numpy as jnp
from jax import lax
from jax.experimental import pallas as pl
from jax.experimental.pallas import tpu as pltpu
from jax.experimental.pallas import tpu_sc as plsc

NUM_EMB = 1_000_000
DIM = 64
BATCH = 16384
HIST = 50
TOTAL = BATCH * HIST

NUM_CORES = 2
NUM_SUBCORES = 16
NW = NUM_CORES * NUM_SUBCORES
PER_W = TOTAL // NW
CH = 128
NSLAB = PER_W // CH

NBUF = 8
LOOK = NBUF // 2

_mesh = plsc.VectorSubcoreMesh(core_axis_name="c", subcore_axis_name="s")


@functools.partial(
    pl.kernel,
    mesh=_mesh,
    out_type=jax.ShapeDtypeStruct((TOTAL, DIM), jnp.float32),
    scratch_types=[
        pltpu.VMEM((NSLAB, CH), jnp.int32),
        pltpu.VMEM((NBUF, CH, DIM), jnp.float32),
        pltpu.SemaphoreType.DMA((NBUF,)),
        pltpu.SemaphoreType.DMA((NBUF,)),
    ],
    compiler_params=pltpu.CompilerParams(use_tc_tiling_on_sc=False),
)
def _gather_rows(ids_hbm, table_hbm, out_hbm, idx_v, rows_v, gsem, ssem):
    wid = lax.axis_index("s") * NUM_CORES + lax.axis_index("c")
    w_base = wid * PER_W

    pltpu.sync_copy(ids_hbm.at[wid], idx_v)

    def fire_gather(s, b):
        pltpu.async_copy(table_hbm.at[idx_v.at[s]], rows_v.at[b], gsem.at[b])

    def drain_gather(s, b):
        pltpu.make_async_copy(
            table_hbm.at[idx_v.at[s]], rows_v.at[b], gsem.at[b]).wait()

    def fire_store(s, b):
        pltpu.async_copy(
            rows_v.at[b], out_hbm.at[pl.ds(w_base + s * CH, CH)], ssem.at[b])

    def wait_store(s, b):
        pltpu.make_async_copy(
            rows_v.at[b], out_hbm.at[pl.ds(w_base + s * CH, CH)], ssem.at[b]).wait()

    for s in range(LOOK):
        fire_gather(s, s)

    for s in range(LOOK):
        drain_gather(s, s)
        fire_store(s, s)
        fire_gather(s + LOOK, (s + LOOK) % NBUF)

    def outer(t, carry):
        for j in range(NBUF):
            s = t * NBUF + j + LOOK
            b = (j + LOOK) % NBUF
            drain_gather(s, b)
            fire_store(s, b)
            wait_store(s - (NBUF - LOOK), j)
            fire_gather(s + LOOK, j)
        return carry

    lax.fori_loop(0, (NSLAB - 2 * LOOK) // NBUF, outer, 0)

    for s in range(NSLAB - LOOK, NSLAB):
        drain_gather(s, s % NBUF)
        fire_store(s, s % NBUF)

    for j in range(NBUF):
        wait_store(NSLAB - NBUF + j, j)


def kernel(input_ids, table):
    ids = input_ids.reshape(NW, NSLAB, CH).astype(jnp.int32)
    out = _gather_rows(ids, table)
    return out.reshape(BATCH, HIST, DIM)
